# Initial kernel scaffold; baseline (speedup 1.0000x reference)
import jax
import jax.numpy as jnp
from jax.experimental import pallas as pl
from jax.experimental.pallas import tpu as pltpu

ALPHA_C = 0.1
K_NN = 4
RADIUS_C = 0.07
H2 = 0.03 ** 2
EPS_C = 1e-12

B, C, N = 16, 3, 2048
ROWS = 256
NBLK = N // ROWS


def _loss_kernel(gt_row_ref, pred_row_ref, pred_ref, rinv_ref, out_ref,
                 colmin_ref):
    b = pl.program_id(0)
    i = pl.program_id(1)

    @pl.when((b == 0) & (i == 0))
    def _init_out():
        out_ref[...] = jnp.zeros((1, 1), jnp.float32)

    @pl.when(i == 0)
    def _init_colmin():
        colmin_ref[...] = jnp.full((1, N), jnp.inf, jnp.float32)

    g = gt_row_ref[0]
    q = pred_row_ref[0]
    p = pred_ref[0]

    d = jnp.zeros((ROWS, N), jnp.float32)
    dpp = jnp.zeros((ROWS, N), jnp.float32)
    for c in range(3):
        pc = p[c:c + 1, :]
        dg = g[:, c:c + 1] - pc
        dq = q[:, c:c + 1] - pc
        d = d + dg * dg
        dpp = dpp + dq * dq

    rinv = rinv_ref[0, 0, 0]
    inv_bn = 1.0 / (B * N)

    rowmin = jnp.min(d, axis=1)
    colmin_ref[...] = jnp.minimum(colmin_ref[...],
                                  jnp.min(d, axis=0, keepdims=True))
    acc = (0.8 * inv_bn) * rinv * jnp.sum(rowmin)

    m = jnp.min(dpp, axis=1, keepdims=True)
    rep = jnp.zeros((), jnp.float32)
    for _ in range(K_NN):
        dpp = jnp.where(dpp == m, jnp.inf, dpp)
        m = jnp.min(dpp, axis=1, keepdims=True)
        d2 = jnp.maximum(m, EPS_C)
        dist = jnp.sqrt(d2)
        w = jnp.exp(-d2 * (1.0 / H2))
        rep = rep + jnp.sum((RADIUS_C - dist) * w)
    acc = acc + (ALPHA_C * inv_bn / K_NN) * rep

    tail = jnp.where(i == NBLK - 1,
                     (0.2 * inv_bn) * rinv * jnp.sum(colmin_ref[...]),
                     0.0)
    out_ref[...] = out_ref[...] + (acc + tail)


def kernel(pred, gt, pcd_radius):
    gt_t = jnp.transpose(gt, (0, 2, 1))
    pred_t = jnp.transpose(pred, (0, 2, 1))
    rinv = (1.0 / pcd_radius).reshape(B, 1, 1)
    out = pl.pallas_call(
        _loss_kernel,
        grid=(B, NBLK),
        in_specs=[
            pl.BlockSpec((1, ROWS, C), lambda b, i: (b, i, 0)),
            pl.BlockSpec((1, ROWS, C), lambda b, i: (b, i, 0)),
            pl.BlockSpec((1, C, N), lambda b, i: (b, 0, 0)),
            pl.BlockSpec((1, 1, 1), lambda b, i: (b, 0, 0)),
        ],
        out_specs=pl.BlockSpec((1, 1), lambda b, i: (0, 0)),
        out_shape=jax.ShapeDtypeStruct((1, 1), jnp.float32),
        scratch_shapes=[pltpu.VMEM((1, N), jnp.float32)],
    )(gt_t, pred_t, pred, rinv)
    return out[0, 0]


# speedup vs baseline: 23.7704x; 23.7704x over previous
import jax
import jax.numpy as jnp
from jax.experimental import pallas as pl
from jax.experimental.pallas import tpu as pltpu

ALPHA_C = 0.1
K_NN = 4
RADIUS_C = 0.07
H2 = 0.03 ** 2
EPS_C = 1e-12

B, C, N = 16, 3, 2048
ROWS = 256
NBLK = N // ROWS


def _loss_kernel(gt_row_ref, pred_row_ref, pred_ref,
                 gtb_row_ref, predb_row_ref, predb_ref,
                 rinv_ref, out_ref, colmin_ref):
    b = pl.program_id(0)
    i = pl.program_id(1)

    @pl.when((b == 0) & (i == 0))
    def _init_out():
        out_ref[...] = jnp.zeros((1, 1), jnp.float32)

    @pl.when(i == 0)
    def _init_colmin():
        colmin_ref[...] = jnp.full((1, N), jnp.inf, jnp.float32)

    g = gt_row_ref[0]
    q = pred_row_ref[0]
    p = pred_ref[0]
    gb = gtb_row_ref[0]
    qb = predb_row_ref[0]
    pb = predb_ref[0]

    g2 = jnp.sum(g * g, axis=1, keepdims=True)
    q2 = jnp.sum(q * q, axis=1, keepdims=True)
    p2 = jnp.sum(p * p, axis=0, keepdims=True)

    ab_g = jnp.zeros((ROWS, N), jnp.float32)
    ab_q = jnp.zeros((ROWS, N), jnp.float32)
    dpp_e = jnp.zeros((ROWS, N), jnp.float32)
    for c in range(3):
        pbc = pb[c:c + 1, :]
        ab_g = ab_g + gb[:, c:c + 1] * pbc
        ab_q = ab_q + qb[:, c:c + 1] * pbc
        dq = q[:, c:c + 1] - p[c:c + 1, :]
        dpp_e = dpp_e + dq * dq

    d_n = (g2 + p2) - 2.0 * ab_g
    dpp_n = (q2 + p2) - 2.0 * ab_q

    rinv = rinv_ref[0, 0, 0]
    inv_bn = 1.0 / (B * N)

    rowmin = jnp.min(d_n, axis=1)
    colmin_ref[...] = jnp.minimum(colmin_ref[...],
                                  jnp.min(d_n, axis=0, keepdims=True))
    acc = (0.8 * inv_bn) * rinv * jnp.sum(rowmin)

    m = jnp.min(dpp_n, axis=1, keepdims=True)
    dpp_n = jnp.where(dpp_n == m, jnp.inf, dpp_n)
    rep = jnp.zeros((), jnp.float32)
    for _ in range(K_NN):
        m = jnp.min(dpp_n, axis=1, keepdims=True)
        sel = dpp_n == m
        e = jnp.min(jnp.where(sel, dpp_e, jnp.inf), axis=1, keepdims=True)
        dpp_n = jnp.where(sel, jnp.inf, dpp_n)
        d2 = jnp.maximum(e, EPS_C)
        dist = jnp.sqrt(d2)
        w = jnp.exp(-d2 * (1.0 / H2))
        rep = rep + jnp.sum((RADIUS_C - dist) * w)
    acc = acc + (ALPHA_C * inv_bn / K_NN) * rep

    tail = jnp.where(i == NBLK - 1,
                     (0.2 * inv_bn) * rinv * jnp.sum(colmin_ref[...]),
                     0.0)
    out_ref[...] = out_ref[...] + (acc + tail)


def kernel(pred, gt, pcd_radius):
    gt_t = jnp.transpose(gt, (0, 2, 1))
    pred_t = jnp.transpose(pred, (0, 2, 1))
    predb = pred.astype(jnp.bfloat16).astype(jnp.float32)
    gtb_t = gt_t.astype(jnp.bfloat16).astype(jnp.float32)
    predb_t = pred_t.astype(jnp.bfloat16).astype(jnp.float32)
    rinv = (1.0 / pcd_radius).reshape(B, 1, 1)
    row_spec = pl.BlockSpec((1, ROWS, C), lambda b, i: (b, i, 0))
    col_spec = pl.BlockSpec((1, C, N), lambda b, i: (b, 0, 0))
    out = pl.pallas_call(
        _loss_kernel,
        grid=(B, NBLK),
        in_specs=[
            row_spec, row_spec, col_spec,
            row_spec, row_spec, col_spec,
            pl.BlockSpec((1, 1, 1), lambda b, i: (b, 0, 0)),
        ],
        out_specs=pl.BlockSpec((1, 1), lambda b, i: (0, 0)),
        out_shape=jax.ShapeDtypeStruct((1, 1), jnp.float32),
        scratch_shapes=[pltpu.VMEM((1, N), jnp.float32)],
    )(gt_t, pred_t, pred, gtb_t, predb_t, predb, rinv)
    return out[0, 0]


# speedup vs baseline: 26.4581x; 1.1131x over previous
import jax
import jax.numpy as jnp
from jax import lax
from jax.experimental import pallas as pl
from jax.experimental.pallas import tpu as pltpu

ALPHA_C = 0.1
K_NN = 4
RADIUS_C = 0.07
H2 = 0.03 ** 2
EPS_C = 1e-12

B, C, N = 16, 3, 2048
C8 = 8
ROWS = 256
NBLK = N // ROWS


def _loss_kernel(gt_row_ref, pred_row_ref, pred_ref,
                 gtb_row_ref, predb_row_ref, predb_ref,
                 rinv_ref, out_ref, colmin_ref):
    b = pl.program_id(0)
    i = pl.program_id(1)

    @pl.when((b == 0) & (i == 0))
    def _init_out():
        out_ref[...] = jnp.zeros((1, 1), jnp.float32)

    @pl.when(i == 0)
    def _init_colmin():
        colmin_ref[...] = jnp.full((1, N), jnp.inf, jnp.float32)

    g = gt_row_ref[0]
    q = pred_row_ref[0]
    p = pred_ref[0]
    gb = gtb_row_ref[0]
    qb = predb_row_ref[0]
    pb = predb_ref[0]

    g2 = jnp.sum(g * g, axis=1, keepdims=True)
    q2 = jnp.sum(q * q, axis=1, keepdims=True)
    p2 = jnp.sum(p * p, axis=0, keepdims=True)

    ab = jnp.dot(jnp.concatenate([gb, qb], axis=0), pb,
                 preferred_element_type=jnp.float32)
    d_n = (g2 + p2) - 2.0 * ab[:ROWS]
    dpp_n = (q2 + p2) - 2.0 * ab[ROWS:]

    ab_e = lax.dot_general(q, p, (((1,), (0,)), ((), ())),
                           precision=lax.Precision.HIGHEST)
    dpp_e = (q2 + p2) - 2.0 * ab_e

    rinv = rinv_ref[0, 0, 0]
    inv_bn = 1.0 / (B * N)

    rowmin = jnp.min(d_n, axis=1)
    colmin_ref[...] = jnp.minimum(colmin_ref[...],
                                  jnp.min(d_n, axis=0, keepdims=True))
    acc = (0.8 * inv_bn) * rinv * jnp.sum(rowmin)

    m = jnp.min(dpp_n, axis=1, keepdims=True)
    dpp_n = jnp.where(dpp_n == m, jnp.inf, dpp_n)
    rep = jnp.zeros((), jnp.float32)
    for _ in range(K_NN):
        m = jnp.min(dpp_n, axis=1, keepdims=True)
        sel = dpp_n == m
        e = jnp.min(jnp.where(sel, dpp_e, jnp.inf), axis=1, keepdims=True)
        dpp_n = jnp.where(sel, jnp.inf, dpp_n)
        d2 = jnp.maximum(e, EPS_C)
        dist = jnp.sqrt(d2)
        w = jnp.exp(-d2 * (1.0 / H2))
        rep = rep + jnp.sum((RADIUS_C - dist) * w)
    acc = acc + (ALPHA_C * inv_bn / K_NN) * rep

    tail = jnp.where(i == NBLK - 1,
                     (0.2 * inv_bn) * rinv * jnp.sum(colmin_ref[...]),
                     0.0)
    out_ref[...] = out_ref[...] + (acc + tail)


def kernel(pred, gt, pcd_radius):
    pad_t = [(0, 0), (0, 0), (0, C8 - C)]
    pad_c = [(0, 0), (0, C8 - C), (0, 0)]
    gt_t = jnp.pad(jnp.transpose(gt, (0, 2, 1)), pad_t)
    pred_t = jnp.pad(jnp.transpose(pred, (0, 2, 1)), pad_t)
    pred_p = jnp.pad(pred, pad_c)
    gtb_t = gt_t.astype(jnp.bfloat16)
    predb_t = pred_t.astype(jnp.bfloat16)
    predb = pred_p.astype(jnp.bfloat16)
    rinv = (1.0 / pcd_radius).reshape(B, 1, 1)
    row_spec = pl.BlockSpec((1, ROWS, C8), lambda b, i: (b, i, 0))
    col_spec = pl.BlockSpec((1, C8, N), lambda b, i: (b, 0, 0))
    out = pl.pallas_call(
        _loss_kernel,
        grid=(B, NBLK),
        in_specs=[
            row_spec, row_spec, col_spec,
            row_spec, row_spec, col_spec,
            pl.BlockSpec((1, 1, 1), lambda b, i: (b, 0, 0)),
        ],
        out_specs=pl.BlockSpec((1, 1), lambda b, i: (0, 0)),
        out_shape=jax.ShapeDtypeStruct((1, 1), jnp.float32),
        scratch_shapes=[pltpu.VMEM((1, N), jnp.float32)],
    )(gt_t, pred_t, pred_p, gtb_t, predb_t, predb, rinv)
    return out[0, 0]
